# baseline (speedup 1.0000x reference)
import functools

import jax
import jax.numpy as jnp
import numpy as np
from jax.experimental import pallas as pl
from jax.experimental.pallas import tpu as pltpu

L_IN = 420
KW = 28
L_OUT = 15
C_OUT = 32
HID = 32
F = L_OUT * C_OUT
P0 = 8
P1 = L_OUT - P0
K0 = P0 * KW
K1 = P1 * KW
N0 = P0 * C_OUT
N1 = P1 * C_OUT
NP = 256


def _net_kernel(x_ref, w0_ref, bb0_ref, w1a_ref,
                w1c_ref, bb1_ref, w1b_ref,
                b1_ref, w2_ref, b2_ref, w3_ref, b3_ref, out_ref):
    xb = x_ref[...].astype(jnp.bfloat16)
    h0 = jnp.dot(xb[:, :K0], w0_ref[...], preferred_element_type=jnp.float32)
    h0 = jnp.maximum(h0 + bb0_ref[...], 0.0).astype(jnp.bfloat16)
    h1 = jnp.dot(xb[:, K0:], w1c_ref[...], preferred_element_type=jnp.float32)
    h1 = jnp.maximum(h1 + bb1_ref[...], 0.0).astype(jnp.bfloat16)
    y = (jnp.dot(h0, w1a_ref[...], preferred_element_type=jnp.float32)
         + jnp.dot(h1, w1b_ref[...], preferred_element_type=jnp.float32))
    y = jnp.maximum(y + b1_ref[...], 0.0).astype(jnp.bfloat16)
    z = jnp.dot(y, w2_ref[...], preferred_element_type=jnp.float32)
    z = jnp.maximum(z + b2_ref[...], 0.0).astype(jnp.bfloat16)
    logit = jnp.dot(z, w3_ref[...], preferred_element_type=jnp.float32) + b3_ref[...]
    out_ref[...] = jax.nn.sigmoid(logit)


def _prep_weights(wc, bc, w1, b1, w2, b2, w3, b3):
    wct = jnp.transpose(wc[:, 0, :]).astype(jnp.float32)
    def blockdiag(npos):
        eye = jnp.eye(npos, dtype=jnp.float32)
        return jnp.einsum('lm,kc->lkmc', eye, wct).reshape(npos * KW, npos * C_OUT)

    w0 = blockdiag(P0).astype(jnp.bfloat16)
    w1c = jnp.pad(blockdiag(P1), ((0, 0), (0, NP - N1))).astype(jnp.bfloat16)
    bb0 = jnp.tile(bc, P0).reshape(1, N0)
    bb1 = jnp.pad(jnp.tile(bc, P1), (0, NP - N1)).reshape(1, NP)
    w1r = jnp.transpose(w1.reshape(HID, C_OUT, L_OUT), (2, 1, 0)).reshape(F, HID)
    w1a = w1r[:N0].astype(jnp.bfloat16)
    w1b = jnp.pad(w1r[N0:], ((0, NP - N1), (0, 0))).astype(jnp.bfloat16)
    b1r = b1.reshape(1, HID)
    w2t = jnp.transpose(w2).astype(jnp.bfloat16)
    b2r = b2.reshape(1, HID)
    w3t = jnp.transpose(w3).astype(jnp.bfloat16)
    b3r = b3.reshape(1, 1)
    return w0, bb0, w1a, w1c, bb1, w1b, b1r, w2t, b2r, w3t, b3r


@jax.jit
def kernel(x, wc, bc, w1, b1, w2, b2, w3, b3):
    B = x.shape[0]
    x_flat = x.reshape(B, L_IN)
    weights = _prep_weights(wc, bc, w1, b1, w2, b2, w3, b3)

    TB = min(1024, max(8, ((B + 7) // 8) * 8))
    Bp = ((B + TB - 1) // TB) * TB
    if Bp != B:
        x_flat = jnp.pad(x_flat, ((0, Bp - B), (0, 0)))
    grid = (Bp // TB,)

    def wspec(shape):
        return pl.BlockSpec(shape, lambda i: (0, 0))

    out = pl.pallas_call(
        _net_kernel,
        out_shape=jax.ShapeDtypeStruct((Bp, 1), jnp.float32),
        grid=grid,
        in_specs=[pl.BlockSpec((TB, L_IN), lambda i: (i, 0)),
                  wspec((K0, N0)), wspec((1, N0)), wspec((N0, HID)),
                  wspec((K1, NP)), wspec((1, NP)), wspec((NP, HID)),
                  wspec((1, HID)), wspec((HID, HID)), wspec((1, HID)),
                  wspec((HID, 1)), wspec((1, 1))],
        out_specs=pl.BlockSpec((TB, 1), lambda i: (i, 0)),
        compiler_params=pltpu.CompilerParams(dimension_semantics=("parallel",)),
    )(x_flat, *weights[:6], *weights[6:])

    return out[:B]
